# Initial kernel scaffold; baseline (speedup 1.0000x reference)
import jax
import jax.numpy as jnp
from jax.experimental import pallas as pl


def kernel(h, e, edge_index, Wn, bn_, We1, be1, We2, be2, Wf1, bf1, Wf2, bf2):
    raise NotImplementedError("write your pallas kernel here")


# speedup vs baseline: 2.4414x; 2.4414x over previous
import functools

import jax
import jax.numpy as jnp
from jax import lax
from jax.experimental import pallas as pl
from jax.experimental.pallas import tpu as pltpu
from jax.experimental.pallas import tpu_sc as plsc

N = 10000
E = 320000
D = 128
DE = 16

NC = 2
NS = 16
LANES = 16

CHUNK = 80
EDGES_PER_TILE = E // (NC * NS)
CHUNKS_PER_TILE = EDGES_PER_TILE // CHUNK
IDX_GROUP = 25
NUM_GROUPS = CHUNKS_PER_TILE // IDX_GROUP
ROWS_PER_TILE = 624
ROWS_TAIL = N - ROWS_PER_TILE * NS


def _ssp(x):
    return jnp.maximum(x, 0.0) + jnp.log1p(jnp.exp(-jnp.abs(x))) - jnp.log(2.0)


def _hp_body(h_ref, w_ref, b_ref, o_ref):
    o_ref[...] = (
        jnp.dot(h_ref[...], w_ref[...], preferred_element_type=jnp.float32)
        + b_ref[...]
    )


def _edge_mlp_body(e_ref, w1_ref, b1_ref, w2_ref, b2_ref, o_ref):
    t = _ssp(
        jnp.dot(e_ref[...], w1_ref[...], preferred_element_type=jnp.float32)
        + b1_ref[...]
    )
    o_ref[...] = _ssp(
        jnp.dot(t, w2_ref[...], preferred_element_type=jnp.float32) + b2_ref[...]
    )


def _final_body(agg_ref, h_ref, w1_ref, b1_ref, w2_ref, b2_ref, o_ref):
    a = agg_ref[0] + agg_ref[1]
    t = _ssp(
        jnp.dot(a, w1_ref[...], preferred_element_type=jnp.float32) + b1_ref[...]
    )
    o_ref[...] = (
        jnp.dot(t, w2_ref[...], preferred_element_type=jnp.float32)
        + b2_ref[...]
        + h_ref[...]
    )


def _hp_call(h, wnT, bn):
    bN = 1000
    return pl.pallas_call(
        _hp_body,
        grid=(N // bN,),
        in_specs=[
            pl.BlockSpec((bN, D), lambda i: (i, 0)),
            pl.BlockSpec((D, D), lambda i: (0, 0)),
            pl.BlockSpec((1, D), lambda i: (0, 0)),
        ],
        out_specs=pl.BlockSpec((bN, D), lambda i: (i, 0)),
        out_shape=jax.ShapeDtypeStruct((N, D), jnp.float32),
    )(h, wnT, bn)


def _edge_mlp_call(e, w1T, b1, w2T, b2):
    bE = 2000
    return pl.pallas_call(
        _edge_mlp_body,
        grid=(E // bE,),
        in_specs=[
            pl.BlockSpec((bE, DE), lambda i: (i, 0)),
            pl.BlockSpec((DE, D), lambda i: (0, 0)),
            pl.BlockSpec((1, D), lambda i: (0, 0)),
            pl.BlockSpec((D, D), lambda i: (0, 0)),
            pl.BlockSpec((1, D), lambda i: (0, 0)),
        ],
        out_specs=pl.BlockSpec((bE, D), lambda i: (i, 0)),
        out_shape=jax.ShapeDtypeStruct((E, D), jnp.float32),
    )(e, w1T, b1, w2T, b2)


def _final_call(agg2, h, w1T, b1, w2T, b2):
    bN = 1000
    return pl.pallas_call(
        _final_body,
        grid=(N // bN,),
        in_specs=[
            pl.BlockSpec((NC, bN, D), lambda i: (0, i, 0)),
            pl.BlockSpec((bN, D), lambda i: (i, 0)),
            pl.BlockSpec((D, D), lambda i: (0, 0)),
            pl.BlockSpec((1, D), lambda i: (0, 0)),
            pl.BlockSpec((D, D), lambda i: (0, 0)),
            pl.BlockSpec((1, D), lambda i: (0, 0)),
        ],
        out_specs=pl.BlockSpec((bN, D), lambda i: (i, 0)),
        out_shape=jax.ShapeDtypeStruct((N, D), jnp.float32),
    )(agg2, h, w1T, b1, w2T, b2)


def _sc_body(hp_hbm, e2_hbm, src_hbm, dst_hbm, z_hbm, out_hbm,
             agg_spmem, src_v, dst_v, rows_v, e2_v, gsem):
    c = lax.axis_index("c")
    s = lax.axis_index("s")
    wid = c * NS + s
    edge_base = wid * EDGES_PER_TILE

    pltpu.sync_copy(
        z_hbm.at[pl.ds(0, ROWS_PER_TILE)],
        agg_spmem.at[pl.ds(s * ROWS_PER_TILE, ROWS_PER_TILE)],
    )

    @pl.when(s == 0)
    def _zero_tail():
        pltpu.sync_copy(
            z_hbm.at[pl.ds(0, ROWS_TAIL)],
            agg_spmem.at[pl.ds(NS * ROWS_PER_TILE, ROWS_TAIL)],
        )

    plsc.subcore_barrier()

    def group(g, carry0):
        pltpu.sync_copy(src_hbm.at[wid, g], src_v)
        pltpu.sync_copy(dst_hbm.at[wid, g], dst_v)

        def chunk(j, carry):
            pltpu.async_copy(hp_hbm.at[src_v.at[j]], rows_v, gsem).wait()
            pltpu.sync_copy(
                e2_hbm.at[pl.ds(edge_base + (g * IDX_GROUP + j) * CHUNK, CHUNK)],
                e2_v,
            )

            def row(r, carry2):
                for l in range(D // LANES):
                    sl = pl.ds(l * LANES, LANES)
                    rows_v[r, sl] = rows_v[r, sl] * e2_v[r, sl]
                return carry2
            lax.fori_loop(0, CHUNK, row, 0)
            pltpu.sync_copy(rows_v, agg_spmem.at[dst_v.at[j]], add=True)
            return carry

        lax.fori_loop(0, IDX_GROUP, chunk, 0)
        return carry0

    lax.fori_loop(0, NUM_GROUPS, group, 0)

    plsc.subcore_barrier()
    pltpu.sync_copy(
        agg_spmem.at[pl.ds(s * ROWS_PER_TILE, ROWS_PER_TILE)],
        out_hbm.at[c, pl.ds(s * ROWS_PER_TILE, ROWS_PER_TILE)],
    )

    @pl.when(s == 0)
    def _out_tail():
        pltpu.sync_copy(
            agg_spmem.at[pl.ds(NS * ROWS_PER_TILE, ROWS_TAIL)],
            out_hbm.at[c, pl.ds(NS * ROWS_PER_TILE, ROWS_TAIL)],
        )


def _sc_call(hp, e2, src_mat, dst_mat, zeros):
    mesh = plsc.VectorSubcoreMesh(core_axis_name="c", subcore_axis_name="s")
    kern = pl.kernel(
        _sc_body,
        out_type=jax.ShapeDtypeStruct((NC, N, D), jnp.float32),
        mesh=mesh,
        scratch_types=[
            pltpu.VMEM_SHARED((N, D), jnp.float32),
            pltpu.VMEM((IDX_GROUP, CHUNK), jnp.int32),
            pltpu.VMEM((IDX_GROUP, CHUNK), jnp.int32),
            pltpu.VMEM((CHUNK, D), jnp.float32),
            pltpu.VMEM((CHUNK, D), jnp.float32),
            pltpu.SemaphoreType.DMA,
        ],
    )
    return kern(hp, e2, src_mat, dst_mat, zeros)


def kernel(h, e, edge_index, Wn, bn_, We1, be1, We2, be2, Wf1, bf1, Wf2, bf2):
    hp = _hp_call(h, Wn.T, bn_.reshape(1, D))
    e2 = _edge_mlp_call(e, We1.T, be1.reshape(1, D), We2.T, be2.reshape(1, D))
    src_mat = edge_index[0].reshape(NC * NS, NUM_GROUPS, IDX_GROUP, CHUNK)
    dst_mat = edge_index[1].reshape(NC * NS, NUM_GROUPS, IDX_GROUP, CHUNK)
    zeros = jnp.zeros((ROWS_PER_TILE, D), jnp.float32)
    agg2 = _sc_call(hp, e2, src_mat, dst_mat, zeros)
    return _final_call(agg2, h, Wf1.T, bf1.reshape(1, D), Wf2.T, bf2.reshape(1, D))
